# baseline (speedup 1.0000x reference)
import numpy as np
import jax
import jax.numpy as jnp
from jax.experimental import pallas as pl
from jax.experimental.pallas import tpu as pltpu

_K = 128
_P = 32
_XB = 256

_KS0 = np.uint32(0)
_KS1 = np.uint32(1)
_KS2 = np.uint32(np.uint32(0x1BD11BDA) ^ _KS0 ^ _KS1)
_ROT_A = (13, 15, 26, 6)
_ROT_B = (17, 29, 16, 24)
_INJ = (
    (_KS1, np.uint32(_KS2 + np.uint32(1))),
    (_KS2, np.uint32(_KS0 + np.uint32(2))),
    (_KS0, np.uint32(_KS1 + np.uint32(3))),
    (_KS1, np.uint32(_KS2 + np.uint32(4))),
    (_KS2, np.uint32(_KS0 + np.uint32(5))),
)


def _rotl(x, r):
    return jax.lax.shift_left(x, np.uint32(r)) | jax.lax.shift_right_logical(
        x, np.uint32(32 - r))


def _threefry2x32_bits(idx):
    rots = (_ROT_A, _ROT_B, _ROT_A, _ROT_B, _ROT_A)
    x1 = idx + _KS1
    x0 = x1
    x1 = _rotl(x1, _ROT_A[0]) ^ x0
    for r in _ROT_A[1:]:
        x0 = x0 + x1
        x1 = _rotl(x1, r) ^ x0
    x0 = x0 + _INJ[0][0]
    x1 = x1 + _INJ[0][1]
    for grp in range(1, 5):
        for r in rots[grp]:
            x0 = x0 + x1
            x1 = _rotl(x1, r) ^ x0
        i0, i1 = _INJ[grp]
        if i0:
            x0 = x0 + i0
        x1 = x1 + i1
    return x0 ^ x1


def _body(x_ref, w1_ref, b1_ref, w2_ref, b2_ref, w3_ref, b3_ref,
          pre_ref, lstd_ref, mm_ref, npf_ref, o_ref):
    blk = pl.program_id(0)
    xcol = x_ref[0]

    h1 = jnp.tanh(xcol * w1_ref[...] + b1_ref[...])
    h2 = jnp.tanh(
        jnp.dot(h1, w2_ref[...], preferred_element_type=jnp.float32)
        + b2_ref[...])
    logits = (jnp.dot(h2, w3_ref[...], preferred_element_type=jnp.float32)
              + b3_ref[...])
    lmax = jnp.max(logits, axis=1, keepdims=True)
    eq = jnp.exp(logits - lmax)
    q = eq / jnp.sum(eq, axis=1, keepdims=True)
    logq = jnp.log(q)

    ix = jax.lax.broadcasted_iota(jnp.int32, (_XB, _P, _K), 0)
    ip = jax.lax.broadcasted_iota(jnp.int32, (_XB, _P, _K), 1)
    ic = jax.lax.broadcasted_iota(jnp.int32, (_XB, _P, _K), 2)
    base = blk * (_XB * _P * _K) + (_K - 1)
    idx = (base + ix * (_P * _K) + ip * _K - ic).astype(jnp.uint32)
    bits = _threefry2x32_bits(idx)
    fbits = jax.lax.shift_right_logical(bits, np.uint32(9)) | np.uint32(0x3F800000)
    tiny = np.float32(np.finfo(np.float32).tiny)
    f = jax.lax.bitcast_convert_type(fbits, jnp.float32) - np.float32(1.0)
    u = jnp.maximum(f, tiny)
    llu = jnp.log(-jnp.log(u))

    t = logq[:, None, :] - llu
    zr = jnp.argmax(t, axis=2).astype(jnp.int32)
    onehot = ic == zr[:, :, None]
    z = (_K - 1) - zr

    pm = pre_ref[...] * np.float32(0.5)
    pmax = jnp.max(pm, axis=1, keepdims=True)
    ep = jnp.exp(pm - pmax)
    logpz_tab = jnp.log(ep / jnp.sum(ep, axis=1, keepdims=True))
    stds_tab = jnp.exp(lstd_ref[...])
    c2pi = np.float32(0.5) * jnp.log(np.float32(2.0 * np.pi))
    crow = ((logpz_tab - jnp.log(stds_tab)) - c2pi) - logq

    zero = np.float32(0.0)
    c_at = jnp.sum(jnp.where(onehot, crow[:, None, :], zero), axis=2)
    s_at = jnp.sum(jnp.where(onehot, stds_tab[:, None, :], zero), axis=2)

    mu = mm_ref[0, 0] * z.astype(jnp.float32)
    d = (xcol - mu) / s_at
    half = np.float32(-0.5)
    log_w = c_at + half * (d * d)

    amax = jnp.max(log_w, axis=1, keepdims=True)
    amax = jnp.where(jnp.isfinite(amax), amax, zero)
    lse = jnp.log(jnp.sum(jnp.exp(log_w - amax), axis=1, keepdims=True)) + amax
    o_ref[0] = -(lse - jnp.log(npf_ref[0, 0]))


def kernel(x, mixture_probs_pre_softmax, mean_multiplier, log_stds,
           W1, b1, W2, b2, W3, b3, num_particles):
    B = x.shape[0]
    nb = B // _XB
    xr = x.reshape(nb, _XB, 1)
    npf = jnp.asarray(num_particles, jnp.float32).reshape(1, 1)

    def _rep(shape):
        return pl.BlockSpec(shape, lambda i: (0,) * len(shape))

    out = pl.pallas_call(
        _body,
        grid=(nb,),
        in_specs=[
            pl.BlockSpec((1, _XB, 1), lambda i: (i, 0, 0)),
            _rep((1, 16)),
            _rep((1, 16)),
            _rep((16, 16)),
            _rep((1, 16)),
            _rep((16, _K)),
            _rep((1, _K)),
            _rep((1, _K)),
            _rep((1, _K)),
            _rep((1, 1)),
            _rep((1, 1)),
        ],
        out_specs=pl.BlockSpec((1, _XB, 1), lambda i: (i, 0, 0)),
        out_shape=jax.ShapeDtypeStruct((nb, _XB, 1), jnp.float32),
        compiler_params=pltpu.CompilerParams(
            dimension_semantics=("parallel",)),
    )(xr, W1, b1.reshape(1, 16), W2, b2.reshape(1, 16), W3[:, ::-1],
      b3[::-1].reshape(1, _K), mixture_probs_pre_softmax[::-1].reshape(1, _K),
      log_stds[::-1].reshape(1, _K), mean_multiplier.reshape(1, 1), npf)
    return out.reshape(B)


# speedup vs baseline: 1.2919x; 1.2919x over previous
import numpy as np
import jax
import jax.numpy as jnp
from jax.experimental import pallas as pl
from jax.experimental.pallas import tpu as pltpu

_K = 128
_P = 32
_XB = 256

_KS0 = np.uint32(0)
_KS1 = np.uint32(1)
_KS2 = np.uint32(np.uint32(0x1BD11BDA) ^ _KS0 ^ _KS1)
_ROT_A = (13, 15, 26, 6)
_ROT_B = (17, 29, 16, 24)
_INJ = (
    (_KS1, np.uint32(_KS2 + np.uint32(1))),
    (_KS2, np.uint32(_KS0 + np.uint32(2))),
    (_KS0, np.uint32(_KS1 + np.uint32(3))),
    (_KS1, np.uint32(_KS2 + np.uint32(4))),
    (_KS2, np.uint32(_KS0 + np.uint32(5))),
)


def _rotl(x, r):
    return jax.lax.shift_left(x, np.uint32(r)) | jax.lax.shift_right_logical(
        x, np.uint32(32 - r))


def _threefry2x32_bits(idx):
    rots = (_ROT_A, _ROT_B, _ROT_A, _ROT_B, _ROT_A)
    x1 = idx + _KS1
    x0 = x1
    x1 = _rotl(x1, _ROT_A[0]) ^ x0
    for r in _ROT_A[1:]:
        x0 = x0 + x1
        x1 = _rotl(x1, r) ^ x0
    x0 = x0 + _INJ[0][0]
    x1 = x1 + _INJ[0][1]
    for grp in range(1, 5):
        for r in rots[grp]:
            x0 = x0 + x1
            x1 = _rotl(x1, r) ^ x0
        i0, i1 = _INJ[grp]
        if i0:
            x0 = x0 + i0
        x1 = x1 + i1
    return x0 ^ x1


def _body(x_ref, sb_ref, w1_ref, b1_ref, w2_ref, b2_ref, w3_ref, b3_ref,
          pre_ref, lstd_ref, mm_ref, npf_ref, o_ref):
    blk = sb_ref[0, 0] + pl.program_id(0)
    xcol = x_ref[0]

    h1 = jnp.tanh(xcol * w1_ref[...] + b1_ref[...])
    h2 = jnp.tanh(
        jnp.dot(h1, w2_ref[...], preferred_element_type=jnp.float32)
        + b2_ref[...])
    logits = (jnp.dot(h2, w3_ref[...], preferred_element_type=jnp.float32)
              + b3_ref[...])
    lmax = jnp.max(logits, axis=1, keepdims=True)
    eq = jnp.exp(logits - lmax)
    q = eq / jnp.sum(eq, axis=1, keepdims=True)
    logq = jnp.log(q)

    ix = jax.lax.broadcasted_iota(jnp.int32, (_XB, _P, _K), 0)
    ip = jax.lax.broadcasted_iota(jnp.int32, (_XB, _P, _K), 1)
    ic = jax.lax.broadcasted_iota(jnp.int32, (_XB, _P, _K), 2)
    base = blk * (_XB * _P * _K) + (_K - 1)
    idx = (base + ix * (_P * _K) + ip * _K - ic).astype(jnp.uint32)
    bits = _threefry2x32_bits(idx)
    fbits = jax.lax.shift_right_logical(bits, np.uint32(9)) | np.uint32(0x3F800000)
    tiny = np.float32(np.finfo(np.float32).tiny)
    f = jax.lax.bitcast_convert_type(fbits, jnp.float32) - np.float32(1.0)
    u = jnp.maximum(f, tiny)
    llu = jnp.log(-jnp.log(u))

    t = logq[:, None, :] - llu
    zr = jnp.argmax(t, axis=2).astype(jnp.int32)
    onehot = ic == zr[:, :, None]
    z = (_K - 1) - zr

    pm = pre_ref[...] * np.float32(0.5)
    pmax = jnp.max(pm, axis=1, keepdims=True)
    ep = jnp.exp(pm - pmax)
    logpz_tab = jnp.log(ep / jnp.sum(ep, axis=1, keepdims=True))
    stds_tab = jnp.exp(lstd_ref[...])
    c2pi = np.float32(0.5) * jnp.log(np.float32(2.0 * np.pi))
    crow = ((logpz_tab - jnp.log(stds_tab)) - c2pi) - logq

    zero = np.float32(0.0)
    c_at = jnp.sum(jnp.where(onehot, crow[:, None, :], zero), axis=2)
    s_at = jnp.sum(jnp.where(onehot, stds_tab[:, None, :], zero), axis=2)

    mu = mm_ref[0, 0] * z.astype(jnp.float32)
    d = (xcol - mu) / s_at
    half = np.float32(-0.5)
    log_w = c_at + half * (d * d)

    amax = jnp.max(log_w, axis=1, keepdims=True)
    amax = jnp.where(jnp.isfinite(amax), amax, zero)
    lse = jnp.log(jnp.sum(jnp.exp(log_w - amax), axis=1, keepdims=True)) + amax
    o_ref[0] = -(lse - jnp.log(npf_ref[0, 0]))


def _run(xr, sb, *params):
    nb = xr.shape[0]

    def _rep(shape):
        return pl.BlockSpec(shape, lambda i: (0,) * len(shape))

    return pl.pallas_call(
        _body,
        grid=(nb,),
        in_specs=[
            pl.BlockSpec((1, _XB, 1), lambda i: (i, 0, 0)),
            _rep((1, 1)),
            _rep((1, 16)),
            _rep((1, 16)),
            _rep((16, 16)),
            _rep((1, 16)),
            _rep((16, _K)),
            _rep((1, _K)),
            _rep((1, _K)),
            _rep((1, _K)),
            _rep((1, 1)),
            _rep((1, 1)),
        ],
        out_specs=pl.BlockSpec((1, _XB, 1), lambda i: (i, 0, 0)),
        out_shape=jax.ShapeDtypeStruct((nb, _XB, 1), jnp.float32),
        compiler_params=pltpu.CompilerParams(
            dimension_semantics=("parallel",)),
    )(xr, sb, *params)


def kernel(x, mixture_probs_pre_softmax, mean_multiplier, log_stds,
           W1, b1, W2, b2, W3, b3, num_particles):
    B = x.shape[0]
    nb = B // _XB
    xr = x.reshape(nb, _XB, 1)
    npf = jnp.asarray(num_particles, jnp.float32).reshape(1, 1)
    params = (W1, b1.reshape(1, 16), W2, b2.reshape(1, 16), W3[:, ::-1],
              b3[::-1].reshape(1, _K),
              mixture_probs_pre_softmax[::-1].reshape(1, _K),
              log_stds[::-1].reshape(1, _K), mean_multiplier.reshape(1, 1),
              npf)

    devs = jax.devices()
    nd = len(devs)
    if nd > 1 and nb % nd == 0:
        mesh = jax.sharding.Mesh(np.array(devs), ("d",))
        spec_x = jax.sharding.PartitionSpec("d", None, None)
        spec_p = jax.sharding.PartitionSpec(None, None)

        def _shard_fn(xr, *params):
            sb = jnp.full((1, 1), jax.lax.axis_index("d") * (nb // nd),
                          jnp.int32)
            return _run(xr, sb, *params)

        out = jax.shard_map(_shard_fn, mesh=mesh,
                            in_specs=(spec_x,) + (spec_p,) * len(params),
                            out_specs=spec_x, check_vma=False)(xr, *params)
    else:
        out = _run(xr, jnp.zeros((1, 1), jnp.int32), *params)
    return out.reshape(B)
